# baseline (speedup 1.0000x reference)
import functools

import jax
import jax.numpy as jnp
from jax.experimental import pallas as pl

_BS, _NN, _D = 2, 512, 128
_NUM_REL = 2

_CONTRACT0 = (((0,), (0,)), ((), ()))


def _split(v):
    vh = v.astype(jnp.bfloat16)
    vl = (v - vh.astype(jnp.float32)).astype(jnp.bfloat16)
    return vh, vl


def _mm3(u, wh, wl):
    uh, ul = _split(u)
    return (jnp.dot(uh, wh, preferred_element_type=jnp.float32)
            + jnp.dot(uh, wl, preferred_element_type=jnp.float32)
            + jnp.dot(ul, wh, preferred_element_type=jnp.float32))


def _agg(a, xh, xl):
    s = jax.lax.dot_general(a, xh, _CONTRACT0,
                            preferred_element_type=jnp.float32)
    return s + jax.lax.dot_general(a, xl, _CONTRACT0,
                                   preferred_element_type=jnp.float32)


def _rgcn_kernel(adj_ref, x_ref, wrel0_ref, wroot0_ref, b0_ref,
                 wrel1_ref, wroot1_ref, b1_ref, out_ref):
    ws = []
    for wrel_ref, wroot_ref, b_ref in ((wrel0_ref, wroot0_ref, b0_ref),
                                       (wrel1_ref, wroot1_ref, b1_ref)):
        ws.append((_split(wroot_ref[...]), _split(wrel_ref[0]),
                   _split(wrel_ref[1]), b_ref[...]))

    for b in range(_BS):
        aug = adj_ref[0, b]
        pun = adj_ref[1, b]
        m1 = aug == 1
        m0 = (pun == 1) & (aug != 1)
        a1 = m1.astype(jnp.bfloat16)
        a0 = m0.astype(jnp.bfloat16)

        inv0 = 1.0 / jnp.maximum(jnp.sum(m0.astype(jnp.float32), axis=0), 1.0)
        inv1 = 1.0 / jnp.maximum(jnp.sum(m1.astype(jnp.float32), axis=0), 1.0)

        x = x_ref[b]
        for (wrh, wrl), (w0h, w0l), (w1h, w1l), bias in ws:
            xh, xl = _split(x)
            h = _mm3(x, wrh, wrl) + bias
            h = h + _mm3(_agg(a0, xh, xl) * inv0[:, None], w0h, w0l)
            h = h + _mm3(_agg(a1, xh, xl) * inv1[:, None], w1h, w1l)
            x = jnp.where(h > 0, h, jnp.exp(jnp.minimum(h, 0.0)) - 1.0)
        out_ref[b] = x


@functools.partial(jax.jit, static_argnames=())
def _run(adj, x, wrel0, wroot0, b0, wrel1, wroot1, b1):
    return pl.pallas_call(
        _rgcn_kernel,
        out_shape=jax.ShapeDtypeStruct((_BS, _NN, _D), jnp.float32),
    )(adj, x, wrel0, wroot0, b0, wrel1, wroot1, b1)


def kernel(feature_list, adj_list, aug_pun_adj, pooled_output, p_nodes_mask,
           o_nodes_mask, W_rel0, W_root0, bias0, W_rel1, W_root1, bias1):
    x = feature_list[0]
    adj = aug_pun_adj.astype(jnp.int32)
    out = _run(adj, x, W_rel0, W_root0, bias0.reshape(1, _D),
               W_rel1, W_root1, bias1.reshape(1, _D))
    return out
